# Initial kernel scaffold; baseline (speedup 1.0000x reference)
import jax
import jax.numpy as jnp
from jax.experimental import pallas as pl


def kernel(x, edge_index, edge_attr, W_enc, b_enc, conv_W, conv_b, ln_g, ln_b, ep_W1, ep_b1, ep_W2, ep_b2, ep_W3, ep_b3, cl_W1, cl_b1, cl_W2, cl_b2):
    raise NotImplementedError("write your pallas kernel here")


# speedup vs baseline: 1.6249x; 1.6249x over previous
import functools

import jax
import jax.numpy as jnp
from jax import lax
from jax.experimental import pallas as pl
from jax.experimental.pallas import tpu as pltpu

N = 10000
E = 640000
D_IN = 22
H = 128
C = 3
L = 3

RB = 1000
NB = N // RB
EB = 5000
NEB = E // EB


def _pre_body(x_ref, we_ref, be_ref, deg_ref, w0_ref, h_ref, hxp_ref, dinv_ref):
    deg = deg_ref[...]
    dinv = jnp.where(deg > 0, lax.rsqrt(deg), 0.0)
    h = jnp.maximum(jnp.dot(x_ref[...], we_ref[...],
                            preferred_element_type=jnp.float32) + be_ref[...], 0.0)
    h_ref[...] = h
    hxp_ref[...] = dinv * jnp.dot(h, w0_ref[...], preferred_element_type=jnp.float32)
    dinv_ref[...] = dinv


def _pre(x, W_enc, b_enc, deg, W0):
    return pl.pallas_call(
        _pre_body,
        grid=(NB,),
        in_specs=[
            pl.BlockSpec((RB, D_IN), lambda i: (i, 0)),
            pl.BlockSpec((D_IN, H), lambda i: (0, 0)),
            pl.BlockSpec((1, H), lambda i: (0, 0)),
            pl.BlockSpec((RB, 1), lambda i: (i, 0)),
            pl.BlockSpec((H, H), lambda i: (0, 0)),
        ],
        out_specs=[
            pl.BlockSpec((RB, H), lambda i: (i, 0)),
            pl.BlockSpec((RB, H), lambda i: (i, 0)),
            pl.BlockSpec((RB, 1), lambda i: (i, 0)),
        ],
        out_shape=[
            jax.ShapeDtypeStruct((N, H), jnp.float32),
            jax.ShapeDtypeStruct((N, H), jnp.float32),
            jax.ShapeDtypeStruct((N, 1), jnp.float32),
        ],
    )(x, W_enc, b_enc, deg, W0)


def _mid_body(h_ref, hxp_ref, ms_ref, dinv_ref, bi_ref, g_ref, bl_ref, wn_ref,
              hn_ref, hxn_ref):
    dinv = dinv_ref[...]
    agg = dinv * (ms_ref[...] + hxp_ref[...]) + bi_ref[...]
    mu = jnp.mean(agg, axis=-1, keepdims=True)
    var = jnp.mean((agg - mu) ** 2, axis=-1, keepdims=True)
    u = (agg - mu) * lax.rsqrt(var + 1e-5) * g_ref[...] + bl_ref[...]
    hn = h_ref[...] + jnp.maximum(u, 0.0)
    hn_ref[...] = hn
    hxn_ref[...] = dinv * jnp.dot(hn, wn_ref[...], preferred_element_type=jnp.float32)


def _mid(h, hxp, msum, dinv, bi, g, bl, Wn):
    return pl.pallas_call(
        _mid_body,
        grid=(NB,),
        in_specs=[
            pl.BlockSpec((RB, H), lambda i: (i, 0)),
            pl.BlockSpec((RB, H), lambda i: (i, 0)),
            pl.BlockSpec((RB, H), lambda i: (i, 0)),
            pl.BlockSpec((RB, 1), lambda i: (i, 0)),
            pl.BlockSpec((1, H), lambda i: (0, 0)),
            pl.BlockSpec((1, H), lambda i: (0, 0)),
            pl.BlockSpec((1, H), lambda i: (0, 0)),
            pl.BlockSpec((H, H), lambda i: (0, 0)),
        ],
        out_specs=[
            pl.BlockSpec((RB, H), lambda i: (i, 0)),
            pl.BlockSpec((RB, H), lambda i: (i, 0)),
        ],
        out_shape=[
            jax.ShapeDtypeStruct((N, H), jnp.float32),
            jax.ShapeDtypeStruct((N, H), jnp.float32),
        ],
    )(h, hxp, msum, dinv, bi, g, bl, Wn)


def _last_body(h_ref, hxp_ref, ms_ref, dinv_ref, bi_ref, g_ref, bl_ref,
               w1a_ref, w1b_ref, b1_ref, clw1_ref, clb1_ref, clw2_ref, clb2_ref,
               a_ref, b_ref, logits_ref, sum_acc, max_acc):
    step = pl.program_id(0)
    dinv = dinv_ref[...]
    agg = dinv * (ms_ref[...] + hxp_ref[...]) + bi_ref[...]
    mu = jnp.mean(agg, axis=-1, keepdims=True)
    var = jnp.mean((agg - mu) ** 2, axis=-1, keepdims=True)
    u = (agg - mu) * lax.rsqrt(var + 1e-5) * g_ref[...] + bl_ref[...]
    hn = h_ref[...] + jnp.maximum(u, 0.0)
    a_ref[...] = jnp.dot(hn, w1a_ref[...], preferred_element_type=jnp.float32) + b1_ref[...]
    b_ref[...] = jnp.dot(hn, w1b_ref[...], preferred_element_type=jnp.float32)

    bsum = jnp.sum(hn, axis=0, keepdims=True)
    bmax = jnp.max(hn, axis=0, keepdims=True)

    @pl.when(step == 0)
    def _():
        sum_acc[...] = bsum
        max_acc[...] = bmax

    @pl.when(step > 0)
    def _():
        sum_acc[...] = sum_acc[...] + bsum
        max_acc[...] = jnp.maximum(max_acc[...], bmax)

    @pl.when(step == NB - 1)
    def _():
        hg = jnp.concatenate([sum_acc[...] * (1.0 / N), max_acc[...]], axis=1)
        z = jnp.maximum(jnp.dot(hg, clw1_ref[...],
                                preferred_element_type=jnp.float32) + clb1_ref[...], 0.0)
        logits_ref[...] = jnp.dot(z, clw2_ref[...],
                                  preferred_element_type=jnp.float32) + clb2_ref[...]


def _last(h, hxp, msum, dinv, bi, g, bl, W1a, W1b, b1, clW1, clb1, clW2, clb2):
    return pl.pallas_call(
        _last_body,
        grid=(NB,),
        in_specs=[
            pl.BlockSpec((RB, H), lambda i: (i, 0)),
            pl.BlockSpec((RB, H), lambda i: (i, 0)),
            pl.BlockSpec((RB, H), lambda i: (i, 0)),
            pl.BlockSpec((RB, 1), lambda i: (i, 0)),
            pl.BlockSpec((1, H), lambda i: (0, 0)),
            pl.BlockSpec((1, H), lambda i: (0, 0)),
            pl.BlockSpec((1, H), lambda i: (0, 0)),
            pl.BlockSpec((H, H), lambda i: (0, 0)),
            pl.BlockSpec((H, H), lambda i: (0, 0)),
            pl.BlockSpec((1, H), lambda i: (0, 0)),
            pl.BlockSpec((2 * H, H), lambda i: (0, 0)),
            pl.BlockSpec((1, H), lambda i: (0, 0)),
            pl.BlockSpec((H, C), lambda i: (0, 0)),
            pl.BlockSpec((1, C), lambda i: (0, 0)),
        ],
        out_specs=[
            pl.BlockSpec((RB, H), lambda i: (i, 0)),
            pl.BlockSpec((RB, H), lambda i: (i, 0)),
            pl.BlockSpec((1, C), lambda i: (0, 0)),
        ],
        out_shape=[
            jax.ShapeDtypeStruct((N, H), jnp.float32),
            jax.ShapeDtypeStruct((N, H), jnp.float32),
            jax.ShapeDtypeStruct((1, C), jnp.float32),
        ],
        scratch_shapes=[
            pltpu.VMEM((1, H), jnp.float32),
            pltpu.VMEM((1, H), jnp.float32),
        ],
    )(h, hxp, msum, dinv, bi, g, bl, W1a, W1b, b1, clW1, clb1, clW2, clb2)


def _ep_body(e1_ref, w2_ref, b2_ref, w3_ref, b3_ref, s_ref):
    e2 = jnp.maximum(jnp.dot(e1_ref[...], w2_ref[...],
                             preferred_element_type=jnp.float32) + b2_ref[...], 0.0)
    z = jnp.dot(e2, w3_ref[...], preferred_element_type=jnp.float32) + b3_ref[...]
    s_ref[...] = 1.0 / (1.0 + jnp.exp(-z))


def _ep_tail(e1, W2, b2, W3, b3):
    return pl.pallas_call(
        _ep_body,
        grid=(NEB,),
        in_specs=[
            pl.BlockSpec((EB, H), lambda i: (i, 0)),
            pl.BlockSpec((H, 32), lambda i: (0, 0)),
            pl.BlockSpec((1, 32), lambda i: (0, 0)),
            pl.BlockSpec((32, 1), lambda i: (0, 0)),
            pl.BlockSpec((1, 1), lambda i: (0, 0)),
        ],
        out_specs=pl.BlockSpec((EB, 1), lambda i: (i, 0)),
        out_shape=jax.ShapeDtypeStruct((E, 1), jnp.float32),
    )(e1, W2, b2, W3, b3)


def kernel(x, edge_index, edge_attr, W_enc, b_enc, conv_W, conv_b, ln_g, ln_b,
           ep_W1, ep_b1, ep_W2, ep_b2, ep_W3, ep_b3, cl_W1, cl_b1, cl_W2, cl_b2):
    src = edge_index[0]
    dst = edge_index[1]
    ew = edge_attr[:, 0]

    deg = jax.ops.segment_sum(ew, dst, num_segments=N) + 1.0
    h, hxp, dinv = _pre(x, W_enc, b_enc.reshape(1, H), deg.reshape(N, 1), conv_W[0])

    for i in range(L):
        msum = jax.ops.segment_sum(ew[:, None] * hxp[src], dst, num_segments=N)
        if i < L - 1:
            h, hxp = _mid(h, hxp, msum, dinv, conv_b[i].reshape(1, H),
                          ln_g[i].reshape(1, H), ln_b[i].reshape(1, H), conv_W[i + 1])
        else:
            A, B, logits = _last(
                h, hxp, msum, dinv, conv_b[i].reshape(1, H),
                ln_g[i].reshape(1, H), ln_b[i].reshape(1, H),
                ep_W1[:H], ep_W1[H:2 * H], ep_b1.reshape(1, H),
                cl_W1, cl_b1.reshape(1, H), cl_W2, cl_b2.reshape(1, C))

    c_row = ep_W1[2 * H]
    e1 = jnp.maximum(A[src] + B[dst] + ew[:, None] * c_row[None, :], 0.0)
    s = _ep_tail(e1, ep_W2, ep_b2.reshape(1, 32), ep_W3, ep_b3.reshape(1, 1))
    return (logits, s[:, 0])


# speedup vs baseline: 8.8416x; 5.4414x over previous
import functools

import jax
import jax.numpy as jnp
from jax import lax
from jax.experimental import pallas as pl
from jax.experimental.pallas import tpu as pltpu
from jax.experimental.pallas import tpu_sc as plsc

N = 10000
E = 640000
D_IN = 22
H = 128
C = 3
L = 3

RB = 1000
NB = N // RB
EB = 5000
NEB = E // EB

SC_NC = 2
SC_NS = 16
NW = SC_NC * SC_NS
CH = 80
NCH = E // (NW * CH)
NSB = 5
SB = NCH // NSB
ZS = 624

_SC_MESH = plsc.VectorSubcoreMesh(
    core_axis_name="c", subcore_axis_name="s",
    num_cores=SC_NC, num_subcores=SC_NS)


def _deg_body(dst_hbm, ew_hbm, out_hbm, dstv, eww, zb, degsh):
    cid = lax.axis_index("c")
    sid = lax.axis_index("s")
    wid = sid * SC_NC + cid

    def zb_body(i, _):
        zb[pl.ds(i * 16, 16)] = jnp.zeros((16,), jnp.float32)
        return 0
    lax.fori_loop(0, 40, zb_body, 0)

    @pl.when(sid < SC_NS - 1)
    def _():
        pltpu.sync_copy(zb.at[pl.ds(0, ZS)], degsh.at[pl.ds(sid * ZS, ZS)])

    @pl.when(sid == SC_NS - 1)
    def _():
        pltpu.sync_copy(zb, degsh.at[pl.ds((SC_NS - 1) * ZS, 640)])

    plsc.subcore_barrier()

    def sbody(sb, _):
        pltpu.sync_copy(dst_hbm.at[wid].at[sb], dstv)
        pltpu.sync_copy(ew_hbm.at[wid].at[sb], eww)

        def body(j, _):
            pltpu.sync_copy(eww.at[j], degsh.at[dstv.at[j]], add=True)
            return 0
        lax.fori_loop(0, SB, body, 0)
        return 0
    lax.fori_loop(0, NSB, sbody, 0)

    plsc.subcore_barrier()

    @pl.when(sid == 0)
    def _():
        pltpu.sync_copy(degsh, out_hbm.at[cid])


_sc_deg = pl.kernel(
    _deg_body,
    out_type=jax.ShapeDtypeStruct((SC_NC, N), jnp.float32),
    mesh=_SC_MESH,
    scratch_types=[
        pltpu.VMEM((SB, CH), jnp.int32),
        pltpu.VMEM((SB, CH), jnp.float32),
        pltpu.VMEM((640,), jnp.float32),
        pltpu.VMEM_SHARED((N,), jnp.float32),
    ],
)


def _scatter_body(hxp_hbm, src_hbm, dst_hbm, ew_hbm, out_hbm,
                  srcv, dstv, eww, rows, accsh, sem):
    cid = lax.axis_index("c")
    sid = lax.axis_index("s")
    wid = sid * SC_NC + cid

    def zr(i, _):
        for f in range(8):
            rows[i, pl.ds(f * 16, 16)] = jnp.zeros((16,), jnp.float32)
        return 0
    lax.fori_loop(0, CH, zr, 0)

    @pl.when(sid < SC_NS - 1)
    def _():
        for k in range(ZS // CH):
            pltpu.sync_copy(rows, accsh.at[pl.ds(sid * ZS + k * CH, CH)])
        pltpu.sync_copy(rows.at[pl.ds(0, ZS % CH)],
                        accsh.at[pl.ds(sid * ZS + ZS - ZS % CH, ZS % CH)])

    @pl.when(sid == SC_NS - 1)
    def _():
        for k in range(8):
            pltpu.sync_copy(rows, accsh.at[pl.ds((SC_NS - 1) * ZS + k * CH, CH)])

    plsc.subcore_barrier()

    def sbody(sb, _):
        pltpu.sync_copy(src_hbm.at[wid].at[sb], srcv)
        pltpu.sync_copy(dst_hbm.at[wid].at[sb], dstv)
        pltpu.sync_copy(ew_hbm.at[wid].at[sb], eww)

        def body(j, _):
            pltpu.async_copy(hxp_hbm.at[srcv.at[j]], rows, sem).wait()

            def eb(g, _):
                w16 = eww[j, pl.ds(g * 16, 16)]
                for ee in range(16):
                    wv = lax.broadcast(w16[ee], (16,))
                    e = g * 16 + ee
                    for f in range(8):
                        rows[e, pl.ds(f * 16, 16)] = rows[e, pl.ds(f * 16, 16)] * wv
                return 0
            lax.fori_loop(0, CH // 16, eb, 0)
            pltpu.sync_copy(rows, accsh.at[dstv.at[j]], add=True)
            return 0
        lax.fori_loop(0, SB, body, 0)
        return 0
    lax.fori_loop(0, NSB, sbody, 0)

    plsc.subcore_barrier()

    @pl.when(sid < SC_NS - 1)
    def _():
        pltpu.sync_copy(accsh.at[pl.ds(sid * ZS, ZS)],
                        out_hbm.at[cid].at[pl.ds(sid * ZS, ZS)])

    @pl.when(sid == SC_NS - 1)
    def _():
        pltpu.sync_copy(accsh.at[pl.ds((SC_NS - 1) * ZS, 640)],
                        out_hbm.at[cid].at[pl.ds((SC_NS - 1) * ZS, 640)])


_sc_scatter = pl.kernel(
    _scatter_body,
    out_type=jax.ShapeDtypeStruct((SC_NC, N, H), jnp.float32),
    mesh=_SC_MESH,
    scratch_types=[
        pltpu.VMEM((SB, CH), jnp.int32),
        pltpu.VMEM((SB, CH), jnp.int32),
        pltpu.VMEM((SB, CH), jnp.float32),
        pltpu.VMEM((CH, H), jnp.float32),
        pltpu.VMEM_SHARED((N, H), jnp.float32),
        pltpu.SemaphoreType.DMA,
    ],
)


def _ep_body(a_hbm, b_hbm, src_hbm, dst_hbm, ew_hbm, c_hbm, e1_hbm,
             srcv, dstv, eww, abuf, bbuf, cbuf, sema, semb):
    cid = lax.axis_index("c")
    sid = lax.axis_index("s")
    wid = sid * SC_NC + cid
    base = wid * NCH
    pltpu.sync_copy(c_hbm, cbuf)
    cv = [cbuf[pl.ds(f * 16, 16)] for f in range(8)]

    def sbody(sb, _):
        pltpu.sync_copy(src_hbm.at[wid].at[sb], srcv)
        pltpu.sync_copy(dst_hbm.at[wid].at[sb], dstv)
        pltpu.sync_copy(ew_hbm.at[wid].at[sb], eww)

        def body(j, _):
            cpa = pltpu.async_copy(a_hbm.at[srcv.at[j]], abuf, sema)
            cpb = pltpu.async_copy(b_hbm.at[dstv.at[j]], bbuf, semb)
            cpa.wait()
            cpb.wait()

            def eb(g, _):
                w16 = eww[j, pl.ds(g * 16, 16)]
                for ee in range(16):
                    wv = lax.broadcast(w16[ee], (16,))
                    e = g * 16 + ee
                    for f in range(8):
                        v = abuf[e, pl.ds(f * 16, 16)] + bbuf[e, pl.ds(f * 16, 16)] \
                            + wv * cv[f]
                        abuf[e, pl.ds(f * 16, 16)] = jnp.maximum(v, 0.0)
                return 0
            lax.fori_loop(0, CH // 16, eb, 0)
            pltpu.sync_copy(abuf, e1_hbm.at[pl.ds((base + sb * SB + j) * CH, CH)])
            return 0
        lax.fori_loop(0, SB, body, 0)
        return 0
    lax.fori_loop(0, NSB, sbody, 0)


_sc_ep = pl.kernel(
    _ep_body,
    out_type=jax.ShapeDtypeStruct((E, H), jnp.float32),
    mesh=_SC_MESH,
    scratch_types=[
        pltpu.VMEM((SB, CH), jnp.int32),
        pltpu.VMEM((SB, CH), jnp.int32),
        pltpu.VMEM((SB, CH), jnp.float32),
        pltpu.VMEM((CH, H), jnp.float32),
        pltpu.VMEM((CH, H), jnp.float32),
        pltpu.VMEM((H,), jnp.float32),
        pltpu.SemaphoreType.DMA,
        pltpu.SemaphoreType.DMA,
    ],
)


def _pre_body(x_ref, we_ref, be_ref, d0_ref, d1_ref, w0_ref, h_ref, hxp_ref, dinv_ref):
    deg = d0_ref[...] + d1_ref[...] + 1.0
    dinv = jnp.where(deg > 0, lax.rsqrt(deg), 0.0)
    h = jnp.maximum(jnp.dot(x_ref[...], we_ref[...],
                            preferred_element_type=jnp.float32) + be_ref[...], 0.0)
    h_ref[...] = h
    hxp_ref[...] = dinv * jnp.dot(h, w0_ref[...], preferred_element_type=jnp.float32)
    dinv_ref[...] = dinv


def _pre(x, W_enc, b_enc, d0, d1, W0):
    return pl.pallas_call(
        _pre_body,
        grid=(NB,),
        in_specs=[
            pl.BlockSpec((RB, D_IN), lambda i: (i, 0)),
            pl.BlockSpec((D_IN, H), lambda i: (0, 0)),
            pl.BlockSpec((1, H), lambda i: (0, 0)),
            pl.BlockSpec((RB, 1), lambda i: (i, 0)),
            pl.BlockSpec((RB, 1), lambda i: (i, 0)),
            pl.BlockSpec((H, H), lambda i: (0, 0)),
        ],
        out_specs=[
            pl.BlockSpec((RB, H), lambda i: (i, 0)),
            pl.BlockSpec((RB, H), lambda i: (i, 0)),
            pl.BlockSpec((RB, 1), lambda i: (i, 0)),
        ],
        out_shape=[
            jax.ShapeDtypeStruct((N, H), jnp.float32),
            jax.ShapeDtypeStruct((N, H), jnp.float32),
            jax.ShapeDtypeStruct((N, 1), jnp.float32),
        ],
    )(x, W_enc, b_enc, d0, d1, W0)


def _mid_body(h_ref, hxp_ref, ms0_ref, ms1_ref, dinv_ref, bi_ref, g_ref, bl_ref,
              wn_ref, hn_ref, hxn_ref):
    dinv = dinv_ref[...]
    agg = dinv * (ms0_ref[...] + ms1_ref[...] + hxp_ref[...]) + bi_ref[...]
    mu = jnp.mean(agg, axis=-1, keepdims=True)
    var = jnp.mean((agg - mu) ** 2, axis=-1, keepdims=True)
    u = (agg - mu) * lax.rsqrt(var + 1e-5) * g_ref[...] + bl_ref[...]
    hn = h_ref[...] + jnp.maximum(u, 0.0)
    hn_ref[...] = hn
    hxn_ref[...] = dinv * jnp.dot(hn, wn_ref[...], preferred_element_type=jnp.float32)


def _mid(h, hxp, ms0, ms1, dinv, bi, g, bl, Wn):
    return pl.pallas_call(
        _mid_body,
        grid=(NB,),
        in_specs=[
            pl.BlockSpec((RB, H), lambda i: (i, 0)),
            pl.BlockSpec((RB, H), lambda i: (i, 0)),
            pl.BlockSpec((RB, H), lambda i: (i, 0)),
            pl.BlockSpec((RB, H), lambda i: (i, 0)),
            pl.BlockSpec((RB, 1), lambda i: (i, 0)),
            pl.BlockSpec((1, H), lambda i: (0, 0)),
            pl.BlockSpec((1, H), lambda i: (0, 0)),
            pl.BlockSpec((1, H), lambda i: (0, 0)),
            pl.BlockSpec((H, H), lambda i: (0, 0)),
        ],
        out_specs=[
            pl.BlockSpec((RB, H), lambda i: (i, 0)),
            pl.BlockSpec((RB, H), lambda i: (i, 0)),
        ],
        out_shape=[
            jax.ShapeDtypeStruct((N, H), jnp.float32),
            jax.ShapeDtypeStruct((N, H), jnp.float32),
        ],
    )(h, hxp, ms0, ms1, dinv, bi, g, bl, Wn)


def _last_body(h_ref, hxp_ref, ms0_ref, ms1_ref, dinv_ref, bi_ref, g_ref, bl_ref,
               w1a_ref, w1b_ref, b1_ref, clw1_ref, clb1_ref, clw2_ref, clb2_ref,
               a_ref, b_ref, logits_ref, sum_acc, max_acc):
    step = pl.program_id(0)
    dinv = dinv_ref[...]
    agg = dinv * (ms0_ref[...] + ms1_ref[...] + hxp_ref[...]) + bi_ref[...]
    mu = jnp.mean(agg, axis=-1, keepdims=True)
    var = jnp.mean((agg - mu) ** 2, axis=-1, keepdims=True)
    u = (agg - mu) * lax.rsqrt(var + 1e-5) * g_ref[...] + bl_ref[...]
    hn = h_ref[...] + jnp.maximum(u, 0.0)
    a_ref[...] = jnp.dot(hn, w1a_ref[...], preferred_element_type=jnp.float32) + b1_ref[...]
    b_ref[...] = jnp.dot(hn, w1b_ref[...], preferred_element_type=jnp.float32)

    bsum = jnp.sum(hn, axis=0, keepdims=True)
    bmax = jnp.max(hn, axis=0, keepdims=True)

    @pl.when(step == 0)
    def _():
        sum_acc[...] = bsum
        max_acc[...] = bmax

    @pl.when(step > 0)
    def _():
        sum_acc[...] = sum_acc[...] + bsum
        max_acc[...] = jnp.maximum(max_acc[...], bmax)

    @pl.when(step == NB - 1)
    def _():
        hg = jnp.concatenate([sum_acc[...] * (1.0 / N), max_acc[...]], axis=1)
        z = jnp.maximum(jnp.dot(hg, clw1_ref[...],
                                preferred_element_type=jnp.float32) + clb1_ref[...], 0.0)
        logits_ref[...] = jnp.dot(z, clw2_ref[...],
                                  preferred_element_type=jnp.float32) + clb2_ref[...]


def _last(h, hxp, ms0, ms1, dinv, bi, g, bl, W1a, W1b, b1, clW1, clb1, clW2, clb2):
    return pl.pallas_call(
        _last_body,
        grid=(NB,),
        in_specs=[
            pl.BlockSpec((RB, H), lambda i: (i, 0)),
            pl.BlockSpec((RB, H), lambda i: (i, 0)),
            pl.BlockSpec((RB, H), lambda i: (i, 0)),
            pl.BlockSpec((RB, H), lambda i: (i, 0)),
            pl.BlockSpec((RB, 1), lambda i: (i, 0)),
            pl.BlockSpec((1, H), lambda i: (0, 0)),
            pl.BlockSpec((1, H), lambda i: (0, 0)),
            pl.BlockSpec((1, H), lambda i: (0, 0)),
            pl.BlockSpec((H, H), lambda i: (0, 0)),
            pl.BlockSpec((H, H), lambda i: (0, 0)),
            pl.BlockSpec((1, H), lambda i: (0, 0)),
            pl.BlockSpec((2 * H, H), lambda i: (0, 0)),
            pl.BlockSpec((1, H), lambda i: (0, 0)),
            pl.BlockSpec((H, C), lambda i: (0, 0)),
            pl.BlockSpec((1, C), lambda i: (0, 0)),
        ],
        out_specs=[
            pl.BlockSpec((RB, H), lambda i: (i, 0)),
            pl.BlockSpec((RB, H), lambda i: (i, 0)),
            pl.BlockSpec((1, C), lambda i: (0, 0)),
        ],
        out_shape=[
            jax.ShapeDtypeStruct((N, H), jnp.float32),
            jax.ShapeDtypeStruct((N, H), jnp.float32),
            jax.ShapeDtypeStruct((1, C), jnp.float32),
        ],
        scratch_shapes=[
            pltpu.VMEM((1, H), jnp.float32),
            pltpu.VMEM((1, H), jnp.float32),
        ],
    )(h, hxp, ms0, ms1, dinv, bi, g, bl, W1a, W1b, b1, clW1, clb1, clW2, clb2)


def _ep_body(e1_ref, w2_ref, b2_ref, w3_ref, b3_ref, s_ref):
    e2 = jnp.maximum(jnp.dot(e1_ref[...], w2_ref[...],
                             preferred_element_type=jnp.float32) + b2_ref[...], 0.0)
    z = jnp.dot(e2, w3_ref[...], preferred_element_type=jnp.float32) + b3_ref[...]
    s_ref[...] = 1.0 / (1.0 + jnp.exp(-z))


def _ep_tail(e1, W2, b2, W3, b3):
    return pl.pallas_call(
        _ep_body,
        grid=(NEB,),
        in_specs=[
            pl.BlockSpec((EB, H), lambda i: (i, 0)),
            pl.BlockSpec((H, 32), lambda i: (0, 0)),
            pl.BlockSpec((1, 32), lambda i: (0, 0)),
            pl.BlockSpec((32, 1), lambda i: (0, 0)),
            pl.BlockSpec((1, 1), lambda i: (0, 0)),
        ],
        out_specs=pl.BlockSpec((EB, 1), lambda i: (i, 0)),
        out_shape=jax.ShapeDtypeStruct((E, 1), jnp.float32),
    )(e1, W2, b2, W3, b3)


def kernel(x, edge_index, edge_attr, W_enc, b_enc, conv_W, conv_b, ln_g, ln_b,
           ep_W1, ep_b1, ep_W2, ep_b2, ep_W3, ep_b3, cl_W1, cl_b1, cl_W2, cl_b2):
    src2d = edge_index[0].reshape(NW, NSB, SB, CH)
    dst2d = edge_index[1].reshape(NW, NSB, SB, CH)
    ew2d = edge_attr[:, 0].reshape(NW, NSB, SB, CH)

    degp = _sc_deg(dst2d, ew2d)
    h, hxp, dinv = _pre(x, W_enc, b_enc.reshape(1, H),
                        degp[0].reshape(N, 1), degp[1].reshape(N, 1), conv_W[0])

    for i in range(L):
        parts = _sc_scatter(hxp, src2d, dst2d, ew2d)
        if i < L - 1:
            h, hxp = _mid(h, hxp, parts[0], parts[1], dinv, conv_b[i].reshape(1, H),
                          ln_g[i].reshape(1, H), ln_b[i].reshape(1, H), conv_W[i + 1])
        else:
            A, B, logits = _last(
                h, hxp, parts[0], parts[1], dinv, conv_b[i].reshape(1, H),
                ln_g[i].reshape(1, H), ln_b[i].reshape(1, H),
                ep_W1[:H], ep_W1[H:2 * H], ep_b1.reshape(1, H),
                cl_W1, cl_b1.reshape(1, H), cl_W2, cl_b2.reshape(1, C))

    e1 = _sc_ep(A, B, src2d, dst2d, ew2d, ep_W1[2 * H])
    s = _ep_tail(e1, ep_W2, ep_b2.reshape(1, 32), ep_W3, ep_b3.reshape(1, 1))
    return (logits, s[:, 0])


# speedup vs baseline: 12.0348x; 1.3612x over previous
import functools

import jax
import jax.numpy as jnp
from jax import lax
from jax.experimental import pallas as pl
from jax.experimental.pallas import tpu as pltpu
from jax.experimental.pallas import tpu_sc as plsc

N = 10000
E = 640000
D_IN = 22
H = 128
C = 3
L = 3

RB = 1000
NB = N // RB
EB = 5000
NEB = E // EB

SC_NC = 2
SC_NS = 16
NW = SC_NC * SC_NS
CH = 80
NCH = E // (NW * CH)
NSB = 5
SB = NCH // NSB
ZS = 624

_SC_MESH = plsc.VectorSubcoreMesh(
    core_axis_name="c", subcore_axis_name="s",
    num_cores=SC_NC, num_subcores=SC_NS)


def _deg_body(dst_hbm, ew_hbm, out_hbm, dstv, eww, zb, degsh):
    cid = lax.axis_index("c")
    sid = lax.axis_index("s")
    wid = sid * SC_NC + cid

    def zb_body(i, _):
        zb[pl.ds(i * 16, 16)] = jnp.zeros((16,), jnp.float32)
        return 0
    lax.fori_loop(0, 40, zb_body, 0)

    @pl.when(sid < SC_NS - 1)
    def _():
        pltpu.sync_copy(zb.at[pl.ds(0, ZS)], degsh.at[pl.ds(sid * ZS, ZS)])

    @pl.when(sid == SC_NS - 1)
    def _():
        pltpu.sync_copy(zb, degsh.at[pl.ds((SC_NS - 1) * ZS, 640)])

    plsc.subcore_barrier()

    def sbody(sb, _):
        pltpu.sync_copy(dst_hbm.at[wid].at[sb], dstv)
        pltpu.sync_copy(ew_hbm.at[wid].at[sb], eww)

        def body(j, _):
            pltpu.sync_copy(eww.at[j], degsh.at[dstv.at[j]], add=True)
            return 0
        lax.fori_loop(0, SB, body, 0)
        return 0
    lax.fori_loop(0, NSB, sbody, 0)

    plsc.subcore_barrier()

    @pl.when(sid == 0)
    def _():
        pltpu.sync_copy(degsh, out_hbm.at[cid])


_sc_deg = pl.kernel(
    _deg_body,
    out_type=jax.ShapeDtypeStruct((SC_NC, N), jnp.float32),
    mesh=_SC_MESH,
    scratch_types=[
        pltpu.VMEM((SB, CH), jnp.int32),
        pltpu.VMEM((SB, CH), jnp.float32),
        pltpu.VMEM((640,), jnp.float32),
        pltpu.VMEM_SHARED((N,), jnp.float32),
    ],
)


def _scatter_body(hxp_hbm, src_hbm, dst_hbm, ew_hbm, out_hbm,
                  srcv, dstv, eww, rows0, rows1, accsh, semg0, semg1):
    cid = lax.axis_index("c")
    sid = lax.axis_index("s")
    wid = sid * SC_NC + cid

    def zr(i, _):
        for f in range(8):
            rows0[i, pl.ds(f * 16, 16)] = jnp.zeros((16,), jnp.float32)
        return 0
    lax.fori_loop(0, CH, zr, 0)

    @pl.when(sid < SC_NS - 1)
    def _():
        for k in range(ZS // CH):
            pltpu.sync_copy(rows0, accsh.at[pl.ds(sid * ZS + k * CH, CH)])
        pltpu.sync_copy(rows0.at[pl.ds(0, ZS % CH)],
                        accsh.at[pl.ds(sid * ZS + ZS - ZS % CH, ZS % CH)])

    @pl.when(sid == SC_NS - 1)
    def _():
        for k in range(8):
            pltpu.sync_copy(rows0, accsh.at[pl.ds((SC_NS - 1) * ZS + k * CH, CH)])

    plsc.subcore_barrier()

    def sbody(sb, _):
        pltpu.sync_copy(src_hbm.at[wid].at[sb], srcv)
        pltpu.sync_copy(dst_hbm.at[wid].at[sb], dstv)
        pltpu.sync_copy(ew_hbm.at[wid].at[sb], eww)
        pltpu.async_copy(hxp_hbm.at[srcv.at[0]], rows0, semg0)
        pltpu.async_copy(hxp_hbm.at[srcv.at[1]], rows1, semg1)

        def body(t, _):
            for k in range(2):
                rbuf = rows0 if k == 0 else rows1
                sg = semg0 if k == 0 else semg1
                j = 2 * t + k
                pltpu.make_async_copy(hxp_hbm.at[srcv.at[j]], rbuf, sg).wait()

                def eb(g, _):
                    w16 = eww[j, pl.ds(g * 16, 16)]
                    for ee in range(16):
                        wv = lax.broadcast(w16[ee], (16,))
                        e = g * 16 + ee
                        for f in range(8):
                            rbuf[e, pl.ds(f * 16, 16)] = \
                                rbuf[e, pl.ds(f * 16, 16)] * wv
                    return 0
                lax.fori_loop(0, CH // 16, eb, 0)
                pltpu.sync_copy(rbuf, accsh.at[dstv.at[j]], add=True)
                nj = j + 2

                @pl.when(nj < SB)
                def _():
                    pltpu.async_copy(hxp_hbm.at[srcv.at[nj]], rbuf, sg)
            return 0
        lax.fori_loop(0, SB // 2, body, 0)
        return 0
    lax.fori_loop(0, NSB, sbody, 0)

    plsc.subcore_barrier()

    @pl.when(sid < SC_NS - 1)
    def _():
        pltpu.sync_copy(accsh.at[pl.ds(sid * ZS, ZS)],
                        out_hbm.at[cid].at[pl.ds(sid * ZS, ZS)])

    @pl.when(sid == SC_NS - 1)
    def _():
        pltpu.sync_copy(accsh.at[pl.ds((SC_NS - 1) * ZS, 640)],
                        out_hbm.at[cid].at[pl.ds((SC_NS - 1) * ZS, 640)])


_sc_scatter = pl.kernel(
    _scatter_body,
    out_type=jax.ShapeDtypeStruct((SC_NC, N, H), jnp.float32),
    mesh=_SC_MESH,
    scratch_types=[
        pltpu.VMEM((SB, CH), jnp.int32),
        pltpu.VMEM((SB, CH), jnp.int32),
        pltpu.VMEM((SB, CH), jnp.float32),
        pltpu.VMEM((CH, H), jnp.float32),
        pltpu.VMEM((CH, H), jnp.float32),
        pltpu.VMEM_SHARED((N, H), jnp.float32),
        pltpu.SemaphoreType.DMA,
        pltpu.SemaphoreType.DMA,
    ],
)


def _ep_body(a_hbm, b_hbm, src_hbm, dst_hbm, ew_hbm, c_hbm, e1_hbm,
             srcv, dstv, eww, abuf0, abuf1, bbuf0, bbuf1, cbuf,
             sema0, sema1, semb0, semb1):
    cid = lax.axis_index("c")
    sid = lax.axis_index("s")
    wid = sid * SC_NC + cid
    base = wid * NCH
    pltpu.sync_copy(c_hbm, cbuf)
    cv = [cbuf[pl.ds(f * 16, 16)] for f in range(8)]

    def sbody(sb, _):
        pltpu.sync_copy(src_hbm.at[wid].at[sb], srcv)
        pltpu.sync_copy(dst_hbm.at[wid].at[sb], dstv)
        pltpu.sync_copy(ew_hbm.at[wid].at[sb], eww)
        pltpu.async_copy(a_hbm.at[srcv.at[0]], abuf0, sema0)
        pltpu.async_copy(b_hbm.at[dstv.at[0]], bbuf0, semb0)
        pltpu.async_copy(a_hbm.at[srcv.at[1]], abuf1, sema1)
        pltpu.async_copy(b_hbm.at[dstv.at[1]], bbuf1, semb1)

        def body(t, _):
            for k in range(2):
                abuf = abuf0 if k == 0 else abuf1
                bbuf = bbuf0 if k == 0 else bbuf1
                sa = sema0 if k == 0 else sema1
                sb_ = semb0 if k == 0 else semb1
                j = 2 * t + k
                pltpu.make_async_copy(a_hbm.at[srcv.at[j]], abuf, sa).wait()
                pltpu.make_async_copy(b_hbm.at[dstv.at[j]], bbuf, sb_).wait()

                def eb(g, _):
                    w16 = eww[j, pl.ds(g * 16, 16)]
                    for ee in range(16):
                        wv = lax.broadcast(w16[ee], (16,))
                        e = g * 16 + ee
                        for f in range(8):
                            v = abuf[e, pl.ds(f * 16, 16)] \
                                + bbuf[e, pl.ds(f * 16, 16)] + wv * cv[f]
                            abuf[e, pl.ds(f * 16, 16)] = jnp.maximum(v, 0.0)
                    return 0
                lax.fori_loop(0, CH // 16, eb, 0)
                pltpu.sync_copy(abuf,
                                e1_hbm.at[pl.ds((base + sb * SB + j) * CH, CH)])
                nj = j + 2

                @pl.when(nj < SB)
                def _():
                    pltpu.async_copy(a_hbm.at[srcv.at[nj]], abuf, sa)
                    pltpu.async_copy(b_hbm.at[dstv.at[nj]], bbuf, sb_)
            return 0
        lax.fori_loop(0, SB // 2, body, 0)
        return 0
    lax.fori_loop(0, NSB, sbody, 0)


_sc_ep = pl.kernel(
    _ep_body,
    out_type=jax.ShapeDtypeStruct((E, H), jnp.float32),
    mesh=_SC_MESH,
    scratch_types=[
        pltpu.VMEM((SB, CH), jnp.int32),
        pltpu.VMEM((SB, CH), jnp.int32),
        pltpu.VMEM((SB, CH), jnp.float32),
        pltpu.VMEM((CH, H), jnp.float32),
        pltpu.VMEM((CH, H), jnp.float32),
        pltpu.VMEM((CH, H), jnp.float32),
        pltpu.VMEM((CH, H), jnp.float32),
        pltpu.VMEM((H,), jnp.float32),
        pltpu.SemaphoreType.DMA,
        pltpu.SemaphoreType.DMA,
        pltpu.SemaphoreType.DMA,
        pltpu.SemaphoreType.DMA,
    ],
)


def _pre_body(x_ref, we_ref, be_ref, d0_ref, d1_ref, w0_ref, h_ref, hxp_ref, dinv_ref):
    deg = d0_ref[...] + d1_ref[...] + 1.0
    dinv = jnp.where(deg > 0, lax.rsqrt(deg), 0.0)
    h = jnp.maximum(jnp.dot(x_ref[...], we_ref[...],
                            preferred_element_type=jnp.float32) + be_ref[...], 0.0)
    h_ref[...] = h
    hxp_ref[...] = dinv * jnp.dot(h, w0_ref[...], preferred_element_type=jnp.float32)
    dinv_ref[...] = dinv


def _pre(x, W_enc, b_enc, d0, d1, W0):
    return pl.pallas_call(
        _pre_body,
        grid=(NB,),
        in_specs=[
            pl.BlockSpec((RB, D_IN), lambda i: (i, 0)),
            pl.BlockSpec((D_IN, H), lambda i: (0, 0)),
            pl.BlockSpec((1, H), lambda i: (0, 0)),
            pl.BlockSpec((RB, 1), lambda i: (i, 0)),
            pl.BlockSpec((RB, 1), lambda i: (i, 0)),
            pl.BlockSpec((H, H), lambda i: (0, 0)),
        ],
        out_specs=[
            pl.BlockSpec((RB, H), lambda i: (i, 0)),
            pl.BlockSpec((RB, H), lambda i: (i, 0)),
            pl.BlockSpec((RB, 1), lambda i: (i, 0)),
        ],
        out_shape=[
            jax.ShapeDtypeStruct((N, H), jnp.float32),
            jax.ShapeDtypeStruct((N, H), jnp.float32),
            jax.ShapeDtypeStruct((N, 1), jnp.float32),
        ],
    )(x, W_enc, b_enc, d0, d1, W0)


def _mid_body(h_ref, hxp_ref, ms0_ref, ms1_ref, dinv_ref, bi_ref, g_ref, bl_ref,
              wn_ref, hn_ref, hxn_ref):
    dinv = dinv_ref[...]
    agg = dinv * (ms0_ref[...] + ms1_ref[...] + hxp_ref[...]) + bi_ref[...]
    mu = jnp.mean(agg, axis=-1, keepdims=True)
    var = jnp.mean((agg - mu) ** 2, axis=-1, keepdims=True)
    u = (agg - mu) * lax.rsqrt(var + 1e-5) * g_ref[...] + bl_ref[...]
    hn = h_ref[...] + jnp.maximum(u, 0.0)
    hn_ref[...] = hn
    hxn_ref[...] = dinv * jnp.dot(hn, wn_ref[...], preferred_element_type=jnp.float32)


def _mid(h, hxp, ms0, ms1, dinv, bi, g, bl, Wn):
    return pl.pallas_call(
        _mid_body,
        grid=(NB,),
        in_specs=[
            pl.BlockSpec((RB, H), lambda i: (i, 0)),
            pl.BlockSpec((RB, H), lambda i: (i, 0)),
            pl.BlockSpec((RB, H), lambda i: (i, 0)),
            pl.BlockSpec((RB, H), lambda i: (i, 0)),
            pl.BlockSpec((RB, 1), lambda i: (i, 0)),
            pl.BlockSpec((1, H), lambda i: (0, 0)),
            pl.BlockSpec((1, H), lambda i: (0, 0)),
            pl.BlockSpec((1, H), lambda i: (0, 0)),
            pl.BlockSpec((H, H), lambda i: (0, 0)),
        ],
        out_specs=[
            pl.BlockSpec((RB, H), lambda i: (i, 0)),
            pl.BlockSpec((RB, H), lambda i: (i, 0)),
        ],
        out_shape=[
            jax.ShapeDtypeStruct((N, H), jnp.float32),
            jax.ShapeDtypeStruct((N, H), jnp.float32),
        ],
    )(h, hxp, ms0, ms1, dinv, bi, g, bl, Wn)


def _last_body(h_ref, hxp_ref, ms0_ref, ms1_ref, dinv_ref, bi_ref, g_ref, bl_ref,
               w1a_ref, w1b_ref, b1_ref, clw1_ref, clb1_ref, clw2_ref, clb2_ref,
               a_ref, b_ref, logits_ref, sum_acc, max_acc):
    step = pl.program_id(0)
    dinv = dinv_ref[...]
    agg = dinv * (ms0_ref[...] + ms1_ref[...] + hxp_ref[...]) + bi_ref[...]
    mu = jnp.mean(agg, axis=-1, keepdims=True)
    var = jnp.mean((agg - mu) ** 2, axis=-1, keepdims=True)
    u = (agg - mu) * lax.rsqrt(var + 1e-5) * g_ref[...] + bl_ref[...]
    hn = h_ref[...] + jnp.maximum(u, 0.0)
    a_ref[...] = jnp.dot(hn, w1a_ref[...], preferred_element_type=jnp.float32) + b1_ref[...]
    b_ref[...] = jnp.dot(hn, w1b_ref[...], preferred_element_type=jnp.float32)

    bsum = jnp.sum(hn, axis=0, keepdims=True)
    bmax = jnp.max(hn, axis=0, keepdims=True)

    @pl.when(step == 0)
    def _():
        sum_acc[...] = bsum
        max_acc[...] = bmax

    @pl.when(step > 0)
    def _():
        sum_acc[...] = sum_acc[...] + bsum
        max_acc[...] = jnp.maximum(max_acc[...], bmax)

    @pl.when(step == NB - 1)
    def _():
        hg = jnp.concatenate([sum_acc[...] * (1.0 / N), max_acc[...]], axis=1)
        z = jnp.maximum(jnp.dot(hg, clw1_ref[...],
                                preferred_element_type=jnp.float32) + clb1_ref[...], 0.0)
        logits_ref[...] = jnp.dot(z, clw2_ref[...],
                                  preferred_element_type=jnp.float32) + clb2_ref[...]


def _last(h, hxp, ms0, ms1, dinv, bi, g, bl, W1a, W1b, b1, clW1, clb1, clW2, clb2):
    return pl.pallas_call(
        _last_body,
        grid=(NB,),
        in_specs=[
            pl.BlockSpec((RB, H), lambda i: (i, 0)),
            pl.BlockSpec((RB, H), lambda i: (i, 0)),
            pl.BlockSpec((RB, H), lambda i: (i, 0)),
            pl.BlockSpec((RB, H), lambda i: (i, 0)),
            pl.BlockSpec((RB, 1), lambda i: (i, 0)),
            pl.BlockSpec((1, H), lambda i: (0, 0)),
            pl.BlockSpec((1, H), lambda i: (0, 0)),
            pl.BlockSpec((1, H), lambda i: (0, 0)),
            pl.BlockSpec((H, H), lambda i: (0, 0)),
            pl.BlockSpec((H, H), lambda i: (0, 0)),
            pl.BlockSpec((1, H), lambda i: (0, 0)),
            pl.BlockSpec((2 * H, H), lambda i: (0, 0)),
            pl.BlockSpec((1, H), lambda i: (0, 0)),
            pl.BlockSpec((H, C), lambda i: (0, 0)),
            pl.BlockSpec((1, C), lambda i: (0, 0)),
        ],
        out_specs=[
            pl.BlockSpec((RB, H), lambda i: (i, 0)),
            pl.BlockSpec((RB, H), lambda i: (i, 0)),
            pl.BlockSpec((1, C), lambda i: (0, 0)),
        ],
        out_shape=[
            jax.ShapeDtypeStruct((N, H), jnp.float32),
            jax.ShapeDtypeStruct((N, H), jnp.float32),
            jax.ShapeDtypeStruct((1, C), jnp.float32),
        ],
        scratch_shapes=[
            pltpu.VMEM((1, H), jnp.float32),
            pltpu.VMEM((1, H), jnp.float32),
        ],
    )(h, hxp, ms0, ms1, dinv, bi, g, bl, W1a, W1b, b1, clW1, clb1, clW2, clb2)


def _ep_body(e1_ref, w2_ref, b2_ref, w3_ref, b3_ref, s_ref):
    e2 = jnp.maximum(jnp.dot(e1_ref[...], w2_ref[...],
                             preferred_element_type=jnp.float32) + b2_ref[...], 0.0)
    z = jnp.dot(e2, w3_ref[...], preferred_element_type=jnp.float32) + b3_ref[...]
    s_ref[...] = 1.0 / (1.0 + jnp.exp(-z))


def _ep_tail(e1, W2, b2, W3, b3):
    return pl.pallas_call(
        _ep_body,
        grid=(NEB,),
        in_specs=[
            pl.BlockSpec((EB, H), lambda i: (i, 0)),
            pl.BlockSpec((H, 32), lambda i: (0, 0)),
            pl.BlockSpec((1, 32), lambda i: (0, 0)),
            pl.BlockSpec((32, 1), lambda i: (0, 0)),
            pl.BlockSpec((1, 1), lambda i: (0, 0)),
        ],
        out_specs=pl.BlockSpec((EB, 1), lambda i: (i, 0)),
        out_shape=jax.ShapeDtypeStruct((E, 1), jnp.float32),
    )(e1, W2, b2, W3, b3)


def kernel(x, edge_index, edge_attr, W_enc, b_enc, conv_W, conv_b, ln_g, ln_b,
           ep_W1, ep_b1, ep_W2, ep_b2, ep_W3, ep_b3, cl_W1, cl_b1, cl_W2, cl_b2):
    src2d = edge_index[0].reshape(NW, NSB, SB, CH)
    dst2d = edge_index[1].reshape(NW, NSB, SB, CH)
    ew2d = edge_attr[:, 0].reshape(NW, NSB, SB, CH)

    degp = _sc_deg(dst2d, ew2d)
    h, hxp, dinv = _pre(x, W_enc, b_enc.reshape(1, H),
                        degp[0].reshape(N, 1), degp[1].reshape(N, 1), conv_W[0])

    for i in range(L):
        parts = _sc_scatter(hxp, src2d, dst2d, ew2d)
        if i < L - 1:
            h, hxp = _mid(h, hxp, parts[0], parts[1], dinv, conv_b[i].reshape(1, H),
                          ln_g[i].reshape(1, H), ln_b[i].reshape(1, H), conv_W[i + 1])
        else:
            A, B, logits = _last(
                h, hxp, parts[0], parts[1], dinv, conv_b[i].reshape(1, H),
                ln_g[i].reshape(1, H), ln_b[i].reshape(1, H),
                ep_W1[:H], ep_W1[H:2 * H], ep_b1.reshape(1, H),
                cl_W1, cl_b1.reshape(1, H), cl_W2, cl_b2.reshape(1, C))

    e1 = _sc_ep(A, B, src2d, dst2d, ew2d, ep_W1[2 * H])
    s = _ep_tail(e1, ep_W2, ep_b2.reshape(1, 32), ep_W3, ep_b3.reshape(1, 1))
    return (logits, s[:, 0])


# speedup vs baseline: 13.2435x; 1.1004x over previous
import functools

import jax
import jax.numpy as jnp
from jax import lax
from jax.experimental import pallas as pl
from jax.experimental.pallas import tpu as pltpu
from jax.experimental.pallas import tpu_sc as plsc

N = 10000
E = 640000
D_IN = 22
H = 128
C = 3
L = 3

RB = 1000
NB = N // RB
EB = 5000
NEB = E // EB

SC_NC = 2
SC_NS = 16
NW = SC_NC * SC_NS
CH = 80
NCH = E // (NW * CH)
NSB = 5
SB = NCH // NSB
ZS = 624

_SC_MESH = plsc.VectorSubcoreMesh(
    core_axis_name="c", subcore_axis_name="s",
    num_cores=SC_NC, num_subcores=SC_NS)


def _deg_body(dst_hbm, ew_hbm, out_hbm, dstv, eww, zb, degsh):
    cid = lax.axis_index("c")
    sid = lax.axis_index("s")
    wid = sid * SC_NC + cid

    def zb_body(i, _):
        zb[pl.ds(i * 16, 16)] = jnp.zeros((16,), jnp.float32)
        return 0
    lax.fori_loop(0, 40, zb_body, 0)

    @pl.when(sid < SC_NS - 1)
    def _():
        pltpu.sync_copy(zb.at[pl.ds(0, ZS)], degsh.at[pl.ds(sid * ZS, ZS)])

    @pl.when(sid == SC_NS - 1)
    def _():
        pltpu.sync_copy(zb, degsh.at[pl.ds((SC_NS - 1) * ZS, 640)])

    plsc.subcore_barrier()

    def sbody(sb, _):
        pltpu.sync_copy(dst_hbm.at[wid].at[sb], dstv)
        pltpu.sync_copy(ew_hbm.at[wid].at[sb], eww)

        def body(j, _):
            pltpu.sync_copy(eww.at[j], degsh.at[dstv.at[j]], add=True)
            return 0
        lax.fori_loop(0, SB, body, 0)
        return 0
    lax.fori_loop(0, NSB, sbody, 0)

    plsc.subcore_barrier()

    @pl.when(sid == 0)
    def _():
        pltpu.sync_copy(degsh, out_hbm.at[cid])


_sc_deg = pl.kernel(
    _deg_body,
    out_type=jax.ShapeDtypeStruct((SC_NC, N), jnp.float32),
    mesh=_SC_MESH,
    scratch_types=[
        pltpu.VMEM((SB, CH), jnp.int32),
        pltpu.VMEM((SB, CH), jnp.float32),
        pltpu.VMEM((640,), jnp.float32),
        pltpu.VMEM_SHARED((N,), jnp.float32),
    ],
)


def _scatter_body(hxp_hbm, src_hbm, dst_hbm, ew_hbm, out_hbm,
                  srcv, dstv, eww, rows0, rows1, accsh, semg0, semg1):
    cid = lax.axis_index("c")
    sid = lax.axis_index("s")
    wid = sid * SC_NC + cid

    def zr(i, _):
        for f in range(8):
            rows0[i, pl.ds(f * 16, 16)] = jnp.zeros((16,), jnp.float32)
        return 0
    lax.fori_loop(0, CH, zr, 0)

    @pl.when(sid < SC_NS - 1)
    def _():
        for k in range(ZS // CH):
            pltpu.sync_copy(rows0, accsh.at[pl.ds(sid * ZS + k * CH, CH)])
        pltpu.sync_copy(rows0.at[pl.ds(0, ZS % CH)],
                        accsh.at[pl.ds(sid * ZS + ZS - ZS % CH, ZS % CH)])

    @pl.when(sid == SC_NS - 1)
    def _():
        for k in range(8):
            pltpu.sync_copy(rows0, accsh.at[pl.ds((SC_NS - 1) * ZS + k * CH, CH)])

    plsc.subcore_barrier()

    def sbody(sb, _):
        pltpu.sync_copy(src_hbm.at[wid].at[sb], srcv)
        pltpu.sync_copy(dst_hbm.at[wid].at[sb], dstv)
        pltpu.sync_copy(ew_hbm.at[wid].at[sb], eww)
        pltpu.async_copy(hxp_hbm.at[srcv.at[0]], rows0, semg0)
        pltpu.async_copy(hxp_hbm.at[srcv.at[1]], rows1, semg1)

        def body(t, _):
            for k in range(2):
                rbuf = rows0 if k == 0 else rows1
                sg = semg0 if k == 0 else semg1
                j = 2 * t + k
                pltpu.make_async_copy(hxp_hbm.at[srcv.at[j]], rbuf, sg).wait()

                def eb(g, _):
                    w16 = eww[j, pl.ds(g * 16, 16)]
                    for ee in range(16):
                        wv = lax.broadcast(w16[ee], (16,))
                        e = g * 16 + ee
                        for f in range(8):
                            rbuf[e, pl.ds(f * 16, 16)] = \
                                rbuf[e, pl.ds(f * 16, 16)] * wv
                    return 0
                lax.fori_loop(0, CH // 16, eb, 0)
                pltpu.sync_copy(rbuf, accsh.at[dstv.at[j]], add=True)
                nj = j + 2

                @pl.when(nj < SB)
                def _():
                    pltpu.async_copy(hxp_hbm.at[srcv.at[nj]], rbuf, sg)
            return 0
        lax.fori_loop(0, SB // 2, body, 0)
        return 0
    lax.fori_loop(0, NSB, sbody, 0)

    plsc.subcore_barrier()

    @pl.when(sid < SC_NS - 1)
    def _():
        pltpu.sync_copy(accsh.at[pl.ds(sid * ZS, ZS)],
                        out_hbm.at[cid].at[pl.ds(sid * ZS, ZS)])

    @pl.when(sid == SC_NS - 1)
    def _():
        pltpu.sync_copy(accsh.at[pl.ds((SC_NS - 1) * ZS, 640)],
                        out_hbm.at[cid].at[pl.ds((SC_NS - 1) * ZS, 640)])


_sc_scatter = pl.kernel(
    _scatter_body,
    out_type=jax.ShapeDtypeStruct((SC_NC, N, H), jnp.float32),
    mesh=_SC_MESH,
    scratch_types=[
        pltpu.VMEM((SB, CH), jnp.int32),
        pltpu.VMEM((SB, CH), jnp.int32),
        pltpu.VMEM((SB, CH), jnp.float32),
        pltpu.VMEM((CH, H), jnp.float32),
        pltpu.VMEM((CH, H), jnp.float32),
        pltpu.VMEM_SHARED((N, H), jnp.float32),
        pltpu.SemaphoreType.DMA,
        pltpu.SemaphoreType.DMA,
    ],
)


KG = 5


def _gather2_body(a_hbm, b_hbm, src_hbm, dst_hbm, as_hbm, bd_hbm,
                  srcv, dstv,
                  ab0, ab1, ab2, ab3, ab4, bb0, bb1, bb2, bb3, bb4,
                  sa0, sa1, sa2, sa3, sa4, sb0, sb1, sb2, sb3, sb4):
    cid = lax.axis_index("c")
    sid = lax.axis_index("s")
    wid = sid * SC_NC + cid
    base = wid * NCH
    abufs = (ab0, ab1, ab2, ab3, ab4)
    bbufs = (bb0, bb1, bb2, bb3, bb4)
    sas = (sa0, sa1, sa2, sa3, sa4)
    sbs = (sb0, sb1, sb2, sb3, sb4)

    def sbody(sb, _):
        pltpu.sync_copy(src_hbm.at[wid].at[sb], srcv)
        pltpu.sync_copy(dst_hbm.at[wid].at[sb], dstv)
        for k in range(KG):
            pltpu.async_copy(a_hbm.at[srcv.at[k]], abufs[k], sas[k])
            pltpu.async_copy(b_hbm.at[dstv.at[k]], bbufs[k], sbs[k])

        def body(t, _):
            for k in range(KG):
                j = KG * t + k
                pltpu.make_async_copy(a_hbm.at[srcv.at[j]], abufs[k], sas[k]).wait()
                pltpu.make_async_copy(b_hbm.at[dstv.at[j]], bbufs[k], sbs[k]).wait()
                row0 = (base + sb * SB + j) * CH
                pltpu.sync_copy(abufs[k], as_hbm.at[pl.ds(row0, CH)])
                pltpu.sync_copy(bbufs[k], bd_hbm.at[pl.ds(row0, CH)])
                nj = j + KG

                @pl.when(nj < SB)
                def _():
                    pltpu.async_copy(a_hbm.at[srcv.at[nj]], abufs[k], sas[k])
                    pltpu.async_copy(b_hbm.at[dstv.at[nj]], bbufs[k], sbs[k])
            return 0
        lax.fori_loop(0, SB // KG, body, 0)
        return 0
    lax.fori_loop(0, NSB, sbody, 0)


_sc_gather2 = pl.kernel(
    _gather2_body,
    out_type=[
        jax.ShapeDtypeStruct((E, H), jnp.float32),
        jax.ShapeDtypeStruct((E, H), jnp.float32),
    ],
    mesh=_SC_MESH,
    scratch_types=(
        [pltpu.VMEM((SB, CH), jnp.int32)] * 2
        + [pltpu.VMEM((CH, H), jnp.float32)] * (2 * KG)
        + [pltpu.SemaphoreType.DMA] * (2 * KG)
    ),
)


def _pre_body(x_ref, we_ref, be_ref, d0_ref, d1_ref, w0_ref, h_ref, hxp_ref, dinv_ref):
    deg = d0_ref[...] + d1_ref[...] + 1.0
    dinv = jnp.where(deg > 0, lax.rsqrt(deg), 0.0)
    h = jnp.maximum(jnp.dot(x_ref[...], we_ref[...],
                            preferred_element_type=jnp.float32) + be_ref[...], 0.0)
    h_ref[...] = h
    hxp_ref[...] = dinv * jnp.dot(h, w0_ref[...], preferred_element_type=jnp.float32)
    dinv_ref[...] = dinv


def _pre(x, W_enc, b_enc, d0, d1, W0):
    return pl.pallas_call(
        _pre_body,
        grid=(NB,),
        in_specs=[
            pl.BlockSpec((RB, D_IN), lambda i: (i, 0)),
            pl.BlockSpec((D_IN, H), lambda i: (0, 0)),
            pl.BlockSpec((1, H), lambda i: (0, 0)),
            pl.BlockSpec((RB, 1), lambda i: (i, 0)),
            pl.BlockSpec((RB, 1), lambda i: (i, 0)),
            pl.BlockSpec((H, H), lambda i: (0, 0)),
        ],
        out_specs=[
            pl.BlockSpec((RB, H), lambda i: (i, 0)),
            pl.BlockSpec((RB, H), lambda i: (i, 0)),
            pl.BlockSpec((RB, 1), lambda i: (i, 0)),
        ],
        out_shape=[
            jax.ShapeDtypeStruct((N, H), jnp.float32),
            jax.ShapeDtypeStruct((N, H), jnp.float32),
            jax.ShapeDtypeStruct((N, 1), jnp.float32),
        ],
    )(x, W_enc, b_enc, d0, d1, W0)


def _mid_body(h_ref, hxp_ref, ms0_ref, ms1_ref, dinv_ref, bi_ref, g_ref, bl_ref,
              wn_ref, hn_ref, hxn_ref):
    dinv = dinv_ref[...]
    agg = dinv * (ms0_ref[...] + ms1_ref[...] + hxp_ref[...]) + bi_ref[...]
    mu = jnp.mean(agg, axis=-1, keepdims=True)
    var = jnp.mean((agg - mu) ** 2, axis=-1, keepdims=True)
    u = (agg - mu) * lax.rsqrt(var + 1e-5) * g_ref[...] + bl_ref[...]
    hn = h_ref[...] + jnp.maximum(u, 0.0)
    hn_ref[...] = hn
    hxn_ref[...] = dinv * jnp.dot(hn, wn_ref[...], preferred_element_type=jnp.float32)


def _mid(h, hxp, ms0, ms1, dinv, bi, g, bl, Wn):
    return pl.pallas_call(
        _mid_body,
        grid=(NB,),
        in_specs=[
            pl.BlockSpec((RB, H), lambda i: (i, 0)),
            pl.BlockSpec((RB, H), lambda i: (i, 0)),
            pl.BlockSpec((RB, H), lambda i: (i, 0)),
            pl.BlockSpec((RB, H), lambda i: (i, 0)),
            pl.BlockSpec((RB, 1), lambda i: (i, 0)),
            pl.BlockSpec((1, H), lambda i: (0, 0)),
            pl.BlockSpec((1, H), lambda i: (0, 0)),
            pl.BlockSpec((1, H), lambda i: (0, 0)),
            pl.BlockSpec((H, H), lambda i: (0, 0)),
        ],
        out_specs=[
            pl.BlockSpec((RB, H), lambda i: (i, 0)),
            pl.BlockSpec((RB, H), lambda i: (i, 0)),
        ],
        out_shape=[
            jax.ShapeDtypeStruct((N, H), jnp.float32),
            jax.ShapeDtypeStruct((N, H), jnp.float32),
        ],
    )(h, hxp, ms0, ms1, dinv, bi, g, bl, Wn)


def _last_body(h_ref, hxp_ref, ms0_ref, ms1_ref, dinv_ref, bi_ref, g_ref, bl_ref,
               w1a_ref, w1b_ref, b1_ref, clw1_ref, clb1_ref, clw2_ref, clb2_ref,
               a_ref, b_ref, logits_ref, sum_acc, max_acc):
    step = pl.program_id(0)
    dinv = dinv_ref[...]
    agg = dinv * (ms0_ref[...] + ms1_ref[...] + hxp_ref[...]) + bi_ref[...]
    mu = jnp.mean(agg, axis=-1, keepdims=True)
    var = jnp.mean((agg - mu) ** 2, axis=-1, keepdims=True)
    u = (agg - mu) * lax.rsqrt(var + 1e-5) * g_ref[...] + bl_ref[...]
    hn = h_ref[...] + jnp.maximum(u, 0.0)
    a_ref[...] = jnp.dot(hn, w1a_ref[...], preferred_element_type=jnp.float32) + b1_ref[...]
    b_ref[...] = jnp.dot(hn, w1b_ref[...], preferred_element_type=jnp.float32)

    bsum = jnp.sum(hn, axis=0, keepdims=True)
    bmax = jnp.max(hn, axis=0, keepdims=True)

    @pl.when(step == 0)
    def _():
        sum_acc[...] = bsum
        max_acc[...] = bmax

    @pl.when(step > 0)
    def _():
        sum_acc[...] = sum_acc[...] + bsum
        max_acc[...] = jnp.maximum(max_acc[...], bmax)

    @pl.when(step == NB - 1)
    def _():
        hg = jnp.concatenate([sum_acc[...] * (1.0 / N), max_acc[...]], axis=1)
        z = jnp.maximum(jnp.dot(hg, clw1_ref[...],
                                preferred_element_type=jnp.float32) + clb1_ref[...], 0.0)
        logits_ref[...] = jnp.dot(z, clw2_ref[...],
                                  preferred_element_type=jnp.float32) + clb2_ref[...]


def _last(h, hxp, ms0, ms1, dinv, bi, g, bl, W1a, W1b, b1, clW1, clb1, clW2, clb2):
    return pl.pallas_call(
        _last_body,
        grid=(NB,),
        in_specs=[
            pl.BlockSpec((RB, H), lambda i: (i, 0)),
            pl.BlockSpec((RB, H), lambda i: (i, 0)),
            pl.BlockSpec((RB, H), lambda i: (i, 0)),
            pl.BlockSpec((RB, H), lambda i: (i, 0)),
            pl.BlockSpec((RB, 1), lambda i: (i, 0)),
            pl.BlockSpec((1, H), lambda i: (0, 0)),
            pl.BlockSpec((1, H), lambda i: (0, 0)),
            pl.BlockSpec((1, H), lambda i: (0, 0)),
            pl.BlockSpec((H, H), lambda i: (0, 0)),
            pl.BlockSpec((H, H), lambda i: (0, 0)),
            pl.BlockSpec((1, H), lambda i: (0, 0)),
            pl.BlockSpec((2 * H, H), lambda i: (0, 0)),
            pl.BlockSpec((1, H), lambda i: (0, 0)),
            pl.BlockSpec((H, C), lambda i: (0, 0)),
            pl.BlockSpec((1, C), lambda i: (0, 0)),
        ],
        out_specs=[
            pl.BlockSpec((RB, H), lambda i: (i, 0)),
            pl.BlockSpec((RB, H), lambda i: (i, 0)),
            pl.BlockSpec((1, C), lambda i: (0, 0)),
        ],
        out_shape=[
            jax.ShapeDtypeStruct((N, H), jnp.float32),
            jax.ShapeDtypeStruct((N, H), jnp.float32),
            jax.ShapeDtypeStruct((1, C), jnp.float32),
        ],
        scratch_shapes=[
            pltpu.VMEM((1, H), jnp.float32),
            pltpu.VMEM((1, H), jnp.float32),
        ],
    )(h, hxp, ms0, ms1, dinv, bi, g, bl, W1a, W1b, b1, clW1, clb1, clW2, clb2)


def _ep_body(a_ref, b_ref, ew_ref, c_ref, w2_ref, b2_ref, w3_ref, b3_ref, s_ref):
    e1 = jnp.maximum(a_ref[...] + b_ref[...] + ew_ref[...] * c_ref[...], 0.0)
    e2 = jnp.maximum(jnp.dot(e1, w2_ref[...],
                             preferred_element_type=jnp.float32) + b2_ref[...], 0.0)
    z = jnp.dot(e2, w3_ref[...], preferred_element_type=jnp.float32) + b3_ref[...]
    s_ref[...] = 1.0 / (1.0 + jnp.exp(-z))


def _ep_tail(asrc, bdst, ew, c_row, W2, b2, W3, b3):
    return pl.pallas_call(
        _ep_body,
        grid=(NEB,),
        in_specs=[
            pl.BlockSpec((EB, H), lambda i: (i, 0)),
            pl.BlockSpec((EB, H), lambda i: (i, 0)),
            pl.BlockSpec((EB, 1), lambda i: (i, 0)),
            pl.BlockSpec((1, H), lambda i: (0, 0)),
            pl.BlockSpec((H, 32), lambda i: (0, 0)),
            pl.BlockSpec((1, 32), lambda i: (0, 0)),
            pl.BlockSpec((32, 1), lambda i: (0, 0)),
            pl.BlockSpec((1, 1), lambda i: (0, 0)),
        ],
        out_specs=pl.BlockSpec((EB, 1), lambda i: (i, 0)),
        out_shape=jax.ShapeDtypeStruct((E, 1), jnp.float32),
    )(asrc, bdst, ew, c_row, W2, b2, W3, b3)


def kernel(x, edge_index, edge_attr, W_enc, b_enc, conv_W, conv_b, ln_g, ln_b,
           ep_W1, ep_b1, ep_W2, ep_b2, ep_W3, ep_b3, cl_W1, cl_b1, cl_W2, cl_b2):
    src2d = edge_index[0].reshape(NW, NSB, SB, CH)
    dst2d = edge_index[1].reshape(NW, NSB, SB, CH)
    ew2d = edge_attr[:, 0].reshape(NW, NSB, SB, CH)

    degp = _sc_deg(dst2d, ew2d)
    h, hxp, dinv = _pre(x, W_enc, b_enc.reshape(1, H),
                        degp[0].reshape(N, 1), degp[1].reshape(N, 1), conv_W[0])

    for i in range(L):
        parts = _sc_scatter(hxp, src2d, dst2d, ew2d)
        if i < L - 1:
            h, hxp = _mid(h, hxp, parts[0], parts[1], dinv, conv_b[i].reshape(1, H),
                          ln_g[i].reshape(1, H), ln_b[i].reshape(1, H), conv_W[i + 1])
        else:
            A, B, logits = _last(
                h, hxp, parts[0], parts[1], dinv, conv_b[i].reshape(1, H),
                ln_g[i].reshape(1, H), ln_b[i].reshape(1, H),
                ep_W1[:H], ep_W1[H:2 * H], ep_b1.reshape(1, H),
                cl_W1, cl_b1.reshape(1, H), cl_W2, cl_b2.reshape(1, C))

    asrc, bdst = _sc_gather2(A, B, src2d, dst2d)
    s = _ep_tail(asrc, bdst, edge_attr, ep_W1[2 * H].reshape(1, H),
                 ep_W2, ep_b2.reshape(1, 32), ep_W3, ep_b3.reshape(1, 1))
    return (logits, s[:, 0])
